# Initial kernel scaffold; baseline (speedup 1.0000x reference)
import jax
import jax.numpy as jnp
from jax.experimental import pallas as pl


def kernel(x, embed_tokens):
    raise NotImplementedError("write your pallas kernel here")


# speedup vs baseline: 1.4802x; 1.4802x over previous
import functools

import jax
import jax.numpy as jnp
from jax import lax
from jax.experimental import pallas as pl
from jax.experimental.pallas import tpu as pltpu
from jax.experimental.pallas import tpu_sc as plsc

VOCAB = 32000
D_MODEL = 4096
N_TOKENS = 4 * 2048

_NUM_CORES = 2
_NUM_SUBCORES = 16
_NW = _NUM_CORES * _NUM_SUBCORES
_ROWS_PER_W = N_TOKENS // _NW
_CHUNK = 8
_N_CHUNKS = _ROWS_PER_W // _CHUNK

_mesh = plsc.VectorSubcoreMesh(core_axis_name="c", subcore_axis_name="s")


@functools.partial(
    pl.kernel,
    mesh=_mesh,
    out_type=jax.ShapeDtypeStruct((N_TOKENS, D_MODEL), jnp.float32),
    scratch_types=[
        pltpu.VMEM((_ROWS_PER_W,), jnp.int32),
        pltpu.VMEM((_CHUNK, D_MODEL), jnp.float32),
        pltpu.SemaphoreType.DMA,
    ],
)
def _embed_gather(x_hbm, tab_hbm, out_hbm, idx_v, rows_v, sem):
    wid = lax.axis_index("s") * _NUM_CORES + lax.axis_index("c")
    base = wid * _ROWS_PER_W
    pltpu.sync_copy(x_hbm.at[pl.ds(base, _ROWS_PER_W)], idx_v)

    def body(i, carry):
        pltpu.async_copy(
            tab_hbm.at[idx_v.at[pl.ds(i * _CHUNK, _CHUNK)]], rows_v, sem
        ).wait()
        pltpu.sync_copy(rows_v, out_hbm.at[pl.ds(base + i * _CHUNK, _CHUNK)])
        return carry

    lax.fori_loop(0, _N_CHUNKS, body, 0, unroll=False)


def kernel(x, embed_tokens):
    flat = x.reshape(-1).astype(jnp.int32)
    out = _embed_gather(flat, embed_tokens)
    return out.reshape(x.shape[0], x.shape[1], D_MODEL)


# speedup vs baseline: 1.7699x; 1.1958x over previous
import functools

import jax
import jax.numpy as jnp
from jax import lax
from jax.experimental import pallas as pl
from jax.experimental.pallas import tpu as pltpu
from jax.experimental.pallas import tpu_sc as plsc

VOCAB = 32000
D_MODEL = 4096
N_TOKENS = 4 * 2048

_NUM_CORES = 2
_NUM_SUBCORES = 16
_NW = _NUM_CORES * _NUM_SUBCORES
_ROWS_PER_W = N_TOKENS // _NW
_CHUNK = 8
_N_CHUNKS = _ROWS_PER_W // _CHUNK

_mesh = plsc.VectorSubcoreMesh(core_axis_name="c", subcore_axis_name="s")


@functools.partial(
    pl.kernel,
    mesh=_mesh,
    out_type=jax.ShapeDtypeStruct((N_TOKENS, D_MODEL), jnp.float32),
    scratch_types=[
        pltpu.VMEM((_ROWS_PER_W,), jnp.int32),
        pltpu.VMEM((_CHUNK, D_MODEL), jnp.float32),
        pltpu.VMEM((_CHUNK, D_MODEL), jnp.float32),
        pltpu.SemaphoreType.DMA,
        pltpu.SemaphoreType.DMA,
        pltpu.SemaphoreType.DMA,
        pltpu.SemaphoreType.DMA,
    ],
)
def _embed_gather(x_hbm, tab_hbm, out_hbm, idx_v, buf0, buf1, g0, g1, o0, o1):
    wid = lax.axis_index("s") * _NUM_CORES + lax.axis_index("c")
    base = wid * _ROWS_PER_W
    pltpu.sync_copy(x_hbm.at[pl.ds(base, _ROWS_PER_W)], idx_v)

    bufs = (buf0, buf1)
    gsems = (g0, g1)
    osems = (o0, o1)

    def gather_desc(i, b):
        return pltpu.make_async_copy(
            tab_hbm.at[idx_v.at[pl.ds(i * _CHUNK, _CHUNK)]], bufs[b], gsems[b]
        )

    def write_desc(i, b):
        return pltpu.make_async_copy(
            bufs[b], out_hbm.at[pl.ds(base + i * _CHUNK, _CHUNK)], osems[b]
        )

    gather_desc(0, 0).start()
    gather_desc(1, 1).start()

    def body(j, carry):
        for b in range(2):
            i = 2 * j + b
            gather_desc(i, b).wait()
            write_desc(i, b).start()
            write_desc(i, b).wait()
            gather_desc(i + 2, b).start()
        return carry

    lax.fori_loop(0, _N_CHUNKS // 2 - 1, body, 0, unroll=False)

    for b in range(2):
        i = _N_CHUNKS - 2 + b
        gather_desc(i, b).wait()
        write_desc(i, b).start()
        write_desc(i, b).wait()


def kernel(x, embed_tokens):
    flat = x.reshape(-1).astype(jnp.int32)
    out = _embed_gather(flat, embed_tokens)
    return out.reshape(x.shape[0], x.shape[1], D_MODEL)
